# baseline (speedup 1.0000x reference)
import functools

import jax
import jax.numpy as jnp
from jax import lax
from jax.experimental import pallas as pl
from jax.experimental.pallas import tpu as pltpu

_FOLD = 4


def _round_up(x, m):
    return (x + m - 1) // m * m


def _stats_kernel(x_ref, w_ref, b_ref, xc_ref, s_ref, *, n_rows, tm, c_out):
    i = pl.program_id(0)
    tf = tm // _FOLD

    def _fold(xb):
        return jnp.concatenate(
            [xb[a * tf:(a + 1) * tf, :] for a in range(_FOLD)], axis=1
        )

    def _head(xf):
        h = jnp.dot(xf, w_ref[...], preferred_element_type=jnp.float32)
        return jnp.maximum(h + b_ref[...], 0.0)

    @pl.when(i < pl.num_programs(0) - 1)
    def _interior():
        xf = _fold(x_ref[...].astype(jnp.bfloat16))
        xc_ref[...] = xf
        h = _head(xf)
        s1 = jnp.sum(h, axis=0, keepdims=True)
        s2 = jnp.sum(h * h, axis=0, keepdims=True)
        s_ref[0] = jnp.concatenate([s1, s2], axis=0)

    @pl.when(i == pl.num_programs(0) - 1)
    def _edge():
        xb = x_ref[...].astype(jnp.bfloat16)
        rr = lax.broadcasted_iota(jnp.int32, xb.shape, 0)
        xb = jnp.where(i * tm + rr < n_rows, xb, jnp.bfloat16(0))
        xf = _fold(xb)
        xc_ref[...] = xf
        h = _head(xf)
        j = lax.broadcasted_iota(jnp.int32, h.shape, 0)
        a = lax.broadcasted_iota(jnp.int32, h.shape, 1) // c_out
        hm = jnp.where(i * tm + a * tf + j < n_rows, h, 0.0)
        s1 = jnp.sum(hm, axis=0, keepdims=True)
        s2 = jnp.sum(hm * hm, axis=0, keepdims=True)
        s_ref[0] = jnp.concatenate([s1, s2], axis=0)


def _apply_kernel(xc_ref, w_ref, b_ref, stats_ref, g_ref, be_ref, o_ref,
                  *, c_out, n_rows, eps):
    tot = jnp.sum(stats_ref[...], axis=0)
    s1 = tot[0:1]
    s2 = tot[1:2]
    s1c = sum(s1[:, a * c_out:(a + 1) * c_out] for a in range(_FOLD))
    s2c = sum(s2[:, a * c_out:(a + 1) * c_out] for a in range(_FOLD))
    inv_n = 1.0 / float(n_rows)
    mean = s1c * inv_n
    var = s2c * inv_n - mean * mean
    scale = lax.rsqrt(var + eps) * g_ref[...]
    shift = be_ref[...] - mean * scale
    scale_f = jnp.concatenate([scale] * _FOLD, axis=1)
    shift_f = jnp.concatenate([shift] * _FOLD, axis=1)

    h = jnp.dot(xc_ref[...], w_ref[...], preferred_element_type=jnp.float32)
    h = jnp.maximum(h + b_ref[...], 0.0)
    y = h * scale_f + shift_f
    tf = y.shape[0]
    for a in range(_FOLD):
        o_ref[a * tf:(a + 1) * tf, :] = y[:, a * c_out:(a + 1) * c_out]


@functools.partial(jax.jit, static_argnames=("eps",))
def _pcloud_head(x, w, b, gamma, beta, *, eps=1e-5):
    n, c_in = x.shape
    c_out = w.shape[1]

    x32 = x.astype(jnp.float32)
    fc_in, fc_out = _FOLD * c_in, _FOLD * c_out

    w_f = jnp.kron(jnp.eye(_FOLD, dtype=jnp.float32),
                   w.astype(jnp.float32)).astype(jnp.bfloat16)
    b_f = jnp.tile(b.astype(jnp.float32).reshape(1, c_out), (1, _FOLD))

    tm = max(8 * _FOLD, min(16384, _round_up(n, 8 * _FOLD)))
    tf = tm // _FOLD
    nb = pl.cdiv(n, tm)
    flops_mm = 2 * n * c_in * c_out
    cparams = pltpu.CompilerParams(dimension_semantics=("parallel",))

    xc, stats = pl.pallas_call(
        functools.partial(_stats_kernel, n_rows=n, tm=tm, c_out=c_out),
        out_shape=(
            jax.ShapeDtypeStruct((nb * tf, fc_in), jnp.bfloat16),
            jax.ShapeDtypeStruct((nb, 2, fc_out), jnp.float32),
        ),
        grid=(nb,),
        in_specs=[
            pl.BlockSpec((tm, c_in), lambda i: (i, 0)),
            pl.BlockSpec((fc_in, fc_out), lambda i: (0, 0)),
            pl.BlockSpec((1, fc_out), lambda i: (0, 0)),
        ],
        out_specs=(
            pl.BlockSpec((tf, fc_in), lambda i: (i, 0)),
            pl.BlockSpec((1, 2, fc_out), lambda i: (i, 0, 0)),
        ),
        compiler_params=cparams,
        cost_estimate=pl.CostEstimate(
            flops=flops_mm,
            transcendentals=0,
            bytes_accessed=x32.size * 4 + nb * tf * fc_in * 2,
        ),
    )(x32, w_f, b_f)

    g2 = gamma.astype(jnp.float32).reshape(1, c_out)
    be2 = beta.astype(jnp.float32).reshape(1, c_out)
    return pl.pallas_call(
        functools.partial(_apply_kernel, c_out=c_out, n_rows=n, eps=eps),
        out_shape=jax.ShapeDtypeStruct((n, c_out), jnp.float32),
        grid=(nb,),
        in_specs=[
            pl.BlockSpec((tf, fc_in), lambda i: (i, 0)),
            pl.BlockSpec((fc_in, fc_out), lambda i: (0, 0)),
            pl.BlockSpec((1, fc_out), lambda i: (0, 0)),
            pl.BlockSpec((nb, 2, fc_out), lambda i: (0, 0, 0)),
            pl.BlockSpec((1, c_out), lambda i: (0, 0)),
            pl.BlockSpec((1, c_out), lambda i: (0, 0)),
        ],
        out_specs=pl.BlockSpec((tm, c_out), lambda i: (i, 0)),
        compiler_params=cparams,
        cost_estimate=pl.CostEstimate(
            flops=flops_mm + 2 * n * c_out,
            transcendentals=0,
            bytes_accessed=nb * tf * fc_in * 2 + n * c_out * 4,
        ),
    )(xc, w_f, b_f, stats, g2, be2)


def kernel(x, w, b, gamma, beta):
    return _pcloud_head(x, w, b, gamma, beta, eps=1e-5)


# speedup vs baseline: 1.0050x; 1.0050x over previous
import functools

import jax
import jax.numpy as jnp
from jax import lax
from jax.experimental import pallas as pl
from jax.experimental.pallas import tpu as pltpu

_FOLD = 4


def _round_up(x, m):
    return (x + m - 1) // m * m


def _stats_kernel(x_ref, w_ref, b_ref, xc_ref, s_ref, *, n_rows, tm, c_out):
    i = pl.program_id(0)
    tf = tm // _FOLD

    def _fold(xb):
        return jnp.concatenate(
            [xb[a * tf:(a + 1) * tf, :] for a in range(_FOLD)], axis=1
        )

    def _head(xf):
        h = jnp.dot(xf, w_ref[...], preferred_element_type=jnp.float32)
        return jnp.maximum(h + b_ref[...], 0.0)

    @pl.when(i < pl.num_programs(0) - 1)
    def _interior():
        xf = _fold(x_ref[...].astype(jnp.bfloat16))
        xc_ref[...] = xf
        h = _head(xf)
        s1 = jnp.sum(h, axis=0, keepdims=True)
        s2 = jnp.sum(h * h, axis=0, keepdims=True)
        s_ref[0] = jnp.concatenate([s1, s2], axis=0)

    @pl.when(i == pl.num_programs(0) - 1)
    def _edge():
        xb = x_ref[...].astype(jnp.bfloat16)
        rr = lax.broadcasted_iota(jnp.int32, xb.shape, 0)
        xb = jnp.where(i * tm + rr < n_rows, xb, jnp.bfloat16(0))
        xf = _fold(xb)
        xc_ref[...] = xf
        h = _head(xf)
        j = lax.broadcasted_iota(jnp.int32, h.shape, 0)
        a = lax.broadcasted_iota(jnp.int32, h.shape, 1) // c_out
        hm = jnp.where(i * tm + a * tf + j < n_rows, h, 0.0)
        s1 = jnp.sum(hm, axis=0, keepdims=True)
        s2 = jnp.sum(hm * hm, axis=0, keepdims=True)
        s_ref[0] = jnp.concatenate([s1, s2], axis=0)


def _apply_kernel(xc_ref, w_ref, b_ref, stats_ref, g_ref, be_ref, o_ref,
                  *, c_out, n_rows, eps):
    tot = jnp.sum(stats_ref[...], axis=0)
    s1 = tot[0:1]
    s2 = tot[1:2]
    s1c = sum(s1[:, a * c_out:(a + 1) * c_out] for a in range(_FOLD))
    s2c = sum(s2[:, a * c_out:(a + 1) * c_out] for a in range(_FOLD))
    inv_n = 1.0 / float(n_rows)
    mean = s1c * inv_n
    var = s2c * inv_n - mean * mean
    scale = lax.rsqrt(var + eps) * g_ref[...]
    shift = be_ref[...] - mean * scale
    scale_f = jnp.concatenate([scale] * _FOLD, axis=1)
    shift_f = jnp.concatenate([shift] * _FOLD, axis=1)

    h = jnp.dot(xc_ref[...], w_ref[...], preferred_element_type=jnp.float32)
    h = jnp.maximum(h + b_ref[...], 0.0)
    y = h * scale_f + shift_f
    tf = y.shape[0]
    for a in range(_FOLD):
        o_ref[a * tf:(a + 1) * tf, :] = y[:, a * c_out:(a + 1) * c_out]


@functools.partial(jax.jit, static_argnames=("eps",))
def _pcloud_head(x, w, b, gamma, beta, *, eps=1e-5):
    n, c_in = x.shape
    c_out = w.shape[1]

    x32 = x.astype(jnp.float32)
    fc_in, fc_out = _FOLD * c_in, _FOLD * c_out

    w_f = jnp.kron(jnp.eye(_FOLD, dtype=jnp.float32),
                   w.astype(jnp.float32)).astype(jnp.bfloat16)
    b_f = jnp.tile(b.astype(jnp.float32).reshape(1, c_out), (1, _FOLD))

    tm = max(8 * _FOLD, min(24576, _round_up(n, 8 * _FOLD)))
    tf = tm // _FOLD
    nb = pl.cdiv(n, tm)
    flops_mm = 2 * n * c_in * c_out
    cparams = pltpu.CompilerParams(dimension_semantics=("parallel",))

    xc, stats = pl.pallas_call(
        functools.partial(_stats_kernel, n_rows=n, tm=tm, c_out=c_out),
        out_shape=(
            jax.ShapeDtypeStruct((nb * tf, fc_in), jnp.bfloat16),
            jax.ShapeDtypeStruct((nb, 2, fc_out), jnp.float32),
        ),
        grid=(nb,),
        in_specs=[
            pl.BlockSpec((tm, c_in), lambda i: (i, 0)),
            pl.BlockSpec((fc_in, fc_out), lambda i: (0, 0)),
            pl.BlockSpec((1, fc_out), lambda i: (0, 0)),
        ],
        out_specs=(
            pl.BlockSpec((tf, fc_in), lambda i: (i, 0)),
            pl.BlockSpec((1, 2, fc_out), lambda i: (i, 0, 0)),
        ),
        compiler_params=cparams,
        cost_estimate=pl.CostEstimate(
            flops=flops_mm,
            transcendentals=0,
            bytes_accessed=x32.size * 4 + nb * tf * fc_in * 2,
        ),
    )(x32, w_f, b_f)

    g2 = gamma.astype(jnp.float32).reshape(1, c_out)
    be2 = beta.astype(jnp.float32).reshape(1, c_out)
    return pl.pallas_call(
        functools.partial(_apply_kernel, c_out=c_out, n_rows=n, eps=eps),
        out_shape=jax.ShapeDtypeStruct((n, c_out), jnp.float32),
        grid=(nb,),
        in_specs=[
            pl.BlockSpec((tf, fc_in), lambda i: (i, 0)),
            pl.BlockSpec((fc_in, fc_out), lambda i: (0, 0)),
            pl.BlockSpec((1, fc_out), lambda i: (0, 0)),
            pl.BlockSpec((nb, 2, fc_out), lambda i: (0, 0, 0)),
            pl.BlockSpec((1, c_out), lambda i: (0, 0)),
            pl.BlockSpec((1, c_out), lambda i: (0, 0)),
        ],
        out_specs=pl.BlockSpec((tm, c_out), lambda i: (i, 0)),
        compiler_params=cparams,
        cost_estimate=pl.CostEstimate(
            flops=flops_mm + 2 * n * c_out,
            transcendentals=0,
            bytes_accessed=nb * tf * fc_in * 2 + n * c_out * 4,
        ),
    )(xc, w_f, b_f, stats, g2, be2)


def kernel(x, w, b, gamma, beta):
    return _pcloud_head(x, w, b, gamma, beta, eps=1e-5)
